# baseline (speedup 1.0000x reference)
import functools

import jax
import jax.numpy as jnp
from jax import lax
from jax.experimental import pallas as pl
from jax.experimental.pallas import tpu as pltpu
from jax.experimental.pallas import tpu_sc as plsc

N = 32768
NSEG = 16
L = 16
NC = 2
NS = 16
NW = NC * NS
CHUNK = N // NW
NEG = -3.0e38
KPRE = 16
KPRE2 = 8


def _sc_body(x_hbm, bounds_hbm, out_hbm, xv, zr, zv, ov, b_ref, dsem):
    wid = lax.axis_index("s") * NC + lax.axis_index("c")
    p0 = (wid * CHUNK).astype(jnp.int32)
    lane = lax.iota(jnp.int32, L)

    cw = p0 >> 10
    wa = jnp.maximum(cw - 1, 0)
    wb = jnp.minimum(cw + 2, NW)

    def dma_fire_early(jc, carry):
        pltpu.make_async_copy(x_hbm.at[pl.ds(jc * CHUNK, CHUNK)],
                              xv.at[pl.ds(jc * CHUNK, CHUNK)], dsem).start()
        return carry

    lax.fori_loop(wa, wb, dma_fire_early, jnp.int32(0))

    pltpu.sync_copy(bounds_hbm, b_ref)
    sv = b_ref[pl.ds(0, L)]
    ev = b_ref[pl.ds(L, L)]

    s_first = jnp.sum(jnp.where(sv <= p0, 1, 0).astype(jnp.int32)) - 1
    s_last = jnp.sum(jnp.where(sv <= p0 + CHUNK - 1, 1, 0).astype(jnp.int32)) - 1
    start_first = jnp.max(jnp.where(lane == s_first, sv, 0))
    end_last = jnp.max(jnp.where(lane == s_last, ev, 0))

    def dma_fire(jc, carry):
        pltpu.make_async_copy(x_hbm.at[pl.ds(jc * CHUNK, CHUNK)],
                              xv.at[pl.ds(jc * CHUNK, CHUNK)], dsem).start()
        return carry

    def dma_drain(jc, carry):
        pltpu.make_async_copy(x_hbm.at[pl.ds(jc * CHUNK, CHUNK)],
                              xv.at[pl.ds(jc * CHUNK, CHUNK)], dsem).wait()
        return carry

    c0 = start_first >> 10
    c1 = (end_last + CHUNK - 1) >> 10
    lax.fori_loop(c0, wa, dma_fire, jnp.int32(0))
    lax.fori_loop(wb, c1, dma_fire, jnp.int32(0))
    lax.fori_loop(jnp.minimum(c0, wa), jnp.maximum(c1, wb), dma_drain,
                  jnp.int32(0))

    def seg_body(s, carry):
        st = jnp.max(jnp.where(lane == s, sv, 0))
        en = jnp.max(jnp.where(lane == s, ev, 0))
        i0 = st >> 4
        i1 = (en + 15) >> 4
        iA = jnp.minimum(i0 + 1, i1)
        iB = jnp.maximum(i1 - 1, iA)

        negf = jnp.full((L,), NEG, jnp.float32)
        for r in range(KPRE):
            zr[pl.ds(r * L, L)] = negf

        def colm_body(j, carry2):
            acc, idxv = carry2
            g = j * L + lane
            v = xv[pl.ds(j * L, L)]
            m = (g >= st) & (g < en) & (v > acc - 1.0)
            plsc.store_scatter(zr, [idxv], v, mask=m)
            return (jnp.maximum(acc, jnp.where(m, v, NEG)),
                    idxv + jnp.where(m, L, 0))

        def colp_body(j, carry2):
            acc, idxv = carry2
            v = xv[pl.ds(j * L, L)]
            m = v > acc - 1.0
            plsc.store_scatter(zr, [idxv], v, mask=m)
            return (jnp.maximum(acc, v), idxv + jnp.where(m, L, 0))

        def colp4_body(t, carry2):
            j = iA + t * 4
            for u in range(4):
                carry2 = colp_body(j + u, carry2)
            return carry2

        nm4 = (iB - iA) >> 2

        def collect():
            carry2 = (negf, lane)
            carry2 = lax.fori_loop(i0, iA, colm_body, carry2)
            carry2 = lax.fori_loop(0, nm4, colp4_body, carry2)
            carry2 = lax.fori_loop(iA + nm4 * 4, iB, colp_body, carry2)
            return lax.fori_loop(iB, i1, colm_body, carry2)

        macc, idxv = collect()
        nvr = jnp.max((idxv - lane) >> 4)
        mx = jnp.max(macc)

        @pl.when(nvr > KPRE)
        def _refill():
            def clr(r, c):
                zr[pl.ds(r * L, L)] = negf
                return c

            lax.fori_loop(0, nvr, clr, jnp.int32(0))
            collect()

        cut = mx - 1.0
        fill = jnp.full((L,), -2.0, jnp.float32)
        for r in range(KPRE2):
            zv[pl.ds(r * L, L)] = fill

        def cpt_body(r, idx2):
            v = zr[pl.ds(r * L, L)]
            m = v > cut
            plsc.store_scatter(zv, [idx2], v - mx, mask=m)
            return idx2 + jnp.where(m, L, 0)

        idx2 = lax.fori_loop(0, nvr, cpt_body, lane)
        nv = jnp.max((idx2 - lane) >> 4)

        @pl.when(nv > KPRE2)
        def _refill2():
            def clr2(r, c):
                zv[pl.ds(r * L, L)] = fill
                return c

            lax.fori_loop(0, nv, clr2, jnp.int32(0))
            lax.fori_loop(0, nvr, cpt_body, lane)

        z0 = zv[pl.ds(0, L)]

        def bis_body(_, lohi):
            lo, hi = lohi
            t = 0.5 * (lo + hi)
            facc = jnp.maximum(z0 - t, 0.0)

            def f_body(j, acc):
                z = zv[pl.ds(j * L + L, L)]
                return acc + jnp.maximum(z - t, 0.0)

            facc = lax.fori_loop(0, nv - 1, f_body, facc)
            f = jnp.sum(facc) - 1.0
            ok = f >= 0.0
            return (jnp.where(ok, t, lo), jnp.where(ok, hi, t))

        lo, _hi = lax.fori_loop(0, 16, bis_body,
                                (jnp.float32(-1.0), jnp.float32(0.0)))

        m0 = z0 > lo

        def ref_body(j, kS):
            k, S = kS
            z = zv[pl.ds(j * L + L, L)]
            m = z > lo
            return (k + m.astype(jnp.int32), S + jnp.where(m, z, 0.0))

        kacc, Sacc = lax.fori_loop(0, nv - 1, ref_body,
                                   (m0.astype(jnp.int32),
                                    jnp.where(m0, z0, 0.0)))
        k = jnp.sum(kacc)
        S = jnp.sum(Sacc)
        kf = jnp.maximum(k, 1).astype(jnp.float32)
        tauv = (jnp.full((L,), S, jnp.float32) - 1.0) / jnp.full((L,), kf,
                                                                 jnp.float32)
        thr = mx + tauv

        ost = jnp.maximum(st, p0)
        oen = jnp.minimum(en, p0 + CHUNK)
        o0 = ost >> 4
        o1 = (oen + 15) >> 4
        oA = jnp.minimum(o0 + 1, o1)
        oB = jnp.maximum(o1 - 1, oA)

        def outm_body(j, c2):
            g = j * L + lane
            v = xv[pl.ds(j * L, L)]
            m = (g >= ost) & (g < oen)
            prev = ov[pl.ds(j * L - p0, L)]
            ov[pl.ds(j * L - p0, L)] = jnp.where(
                m, jnp.maximum(v - thr, 0.0), prev)
            return c2

        def outp_body(j, c2):
            v = xv[pl.ds(j * L, L)]
            ov[pl.ds(j * L - p0, L)] = jnp.maximum(v - thr, 0.0)
            return c2

        def outp4_body(t, c2):
            j = oA + t * 4
            for u in range(4):
                v = xv[pl.ds(j * L + u * L, L)]
                ov[pl.ds(j * L + u * L - p0, L)] = jnp.maximum(v - thr, 0.0)
            return c2

        no4 = (oB - oA) >> 2
        lax.fori_loop(o0, oA, outm_body, jnp.int32(0))
        lax.fori_loop(0, no4, outp4_body, jnp.int32(0))
        lax.fori_loop(oA + no4 * 4, oB, outp_body, jnp.int32(0))
        lax.fori_loop(oB, o1, outm_body, jnp.int32(0))
        return carry

    lax.fori_loop(s_first, s_last + 1, seg_body, jnp.int32(0))
    pltpu.sync_copy(ov, out_hbm.at[pl.ds(p0, CHUNK)])


@jax.jit
def _run(x, bounds):
    mesh = plsc.VectorSubcoreMesh(core_axis_name="c", subcore_axis_name="s")
    f = functools.partial(
        pl.kernel,
        mesh=mesh,
        out_type=jax.ShapeDtypeStruct((N,), jnp.float32),
        scratch_types=[
            pltpu.VMEM((N,), jnp.float32),
            pltpu.VMEM((N,), jnp.float32),
            pltpu.VMEM((N + L,), jnp.float32),
            pltpu.VMEM((CHUNK,), jnp.float32),
            pltpu.VMEM((2 * L,), jnp.int32),
            pltpu.SemaphoreType.DMA,
        ],
        compiler_params=pltpu.CompilerParams(needs_layout_passes=False,
                                             disable_bounds_checks=True),
    )(_sc_body)
    return f(x, bounds)


def kernel(x, batch):
    batch = batch.astype(jnp.int32)
    ids = jnp.arange(NSEG + 1, dtype=jnp.int32)
    cnt = jnp.sum((ids[:, None] > batch[None, :]).astype(jnp.int32), axis=1)
    bounds = jnp.concatenate([cnt[:NSEG], cnt[1:]]).astype(jnp.int32)
    return _run(x.astype(jnp.float32), bounds)


# speedup vs baseline: 1.0928x; 1.0928x over previous
import functools

import jax
import jax.numpy as jnp
from jax import lax
from jax.experimental import pallas as pl
from jax.experimental.pallas import tpu as pltpu
from jax.experimental.pallas import tpu_sc as plsc

N = 32768
NSEG = 16
L = 16
NC = 2
NS = 16
NW = NC * NS
CHUNK = N // NW
NEG = -3.0e38
KPRE = 8


def _sc_body(x_hbm, bounds_hbm, out_hbm, xv, zv, ov, b_ref, dsem):
    wid = lax.axis_index("s") * NC + lax.axis_index("c")
    p0 = (wid * CHUNK).astype(jnp.int32)
    lane = lax.iota(jnp.int32, L)

    cw = p0 >> 10
    wa = jnp.maximum(cw - 1, 0)
    wb = jnp.minimum(cw + 2, NW)

    def dma_fire_early(jc, carry):
        pltpu.make_async_copy(x_hbm.at[pl.ds(jc * CHUNK, CHUNK)],
                              xv.at[pl.ds(jc * CHUNK, CHUNK)], dsem).start()
        return carry

    lax.fori_loop(wa, wb, dma_fire_early, jnp.int32(0))

    pltpu.sync_copy(bounds_hbm, b_ref)
    sv = b_ref[pl.ds(0, L)]
    ev = b_ref[pl.ds(L, L)]

    s_first = jnp.sum(jnp.where(sv <= p0, 1, 0).astype(jnp.int32)) - 1
    s_last = jnp.sum(jnp.where(sv <= p0 + CHUNK - 1, 1, 0).astype(jnp.int32)) - 1
    start_first = jnp.max(jnp.where(lane == s_first, sv, 0))
    end_last = jnp.max(jnp.where(lane == s_last, ev, 0))

    def dma_fire(jc, carry):
        pltpu.make_async_copy(x_hbm.at[pl.ds(jc * CHUNK, CHUNK)],
                              xv.at[pl.ds(jc * CHUNK, CHUNK)], dsem).start()
        return carry

    def dma_drain(jc, carry):
        pltpu.make_async_copy(x_hbm.at[pl.ds(jc * CHUNK, CHUNK)],
                              xv.at[pl.ds(jc * CHUNK, CHUNK)], dsem).wait()
        return carry

    c0 = start_first >> 10
    c1 = (end_last + CHUNK - 1) >> 10
    lax.fori_loop(c0, wa, dma_fire, jnp.int32(0))
    lax.fori_loop(wb, c1, dma_fire, jnp.int32(0))
    lax.fori_loop(jnp.minimum(c0, wa), jnp.maximum(c1, wb), dma_drain,
                  jnp.int32(0))

    def seg_body(s, carry):
        st = jnp.max(jnp.where(lane == s, sv, 0))
        en = jnp.max(jnp.where(lane == s, ev, 0))
        i0 = st >> 4
        i1 = (en + 15) >> 4
        iA = jnp.minimum(i0 + 1, i1)
        iB = jnp.maximum(i1 - 1, iA)

        def maxm_body(j, acc):
            g = j * L + lane
            v = xv[pl.ds(j * L, L)]
            m = (g >= st) & (g < en)
            return jnp.maximum(acc, jnp.where(m, v, NEG))

        def maxp_body(j, acc):
            return jnp.maximum(acc, xv[pl.ds(j * L, L)])

        def maxp4_body(t, acc):
            j = iA + t * 4
            a = jnp.maximum(xv[pl.ds(j * L, L)], xv[pl.ds(j * L + L, L)])
            b = jnp.maximum(xv[pl.ds(j * L + 2 * L, L)],
                            xv[pl.ds(j * L + 3 * L, L)])
            return jnp.maximum(acc, jnp.maximum(a, b))

        nm4 = (iB - iA) >> 2
        macc = jnp.full((L,), NEG, jnp.float32)
        macc = lax.fori_loop(i0, iA, maxm_body, macc)
        macc = lax.fori_loop(0, nm4, maxp4_body, macc)
        macc = lax.fori_loop(iA + nm4 * 4, iB, maxp_body, macc)
        macc = lax.fori_loop(iB, i1, maxm_body, macc)
        mx = jnp.max(macc)

        cut = mx - 1.0
        fill = jnp.full((L,), -2.0, jnp.float32)
        for r in range(KPRE):
            zv[pl.ds(r * L, L)] = fill

        def cmpm_body(j, idxv):
            g = j * L + lane
            v = xv[pl.ds(j * L, L)]
            m = (g >= st) & (g < en) & (v > cut)
            plsc.store_scatter(zv, [idxv], v - mx, mask=m)
            return idxv + jnp.where(m, L, 0)

        def cmpp_body(j, idxv):
            v = xv[pl.ds(j * L, L)]
            m = v > cut
            plsc.store_scatter(zv, [idxv], v - mx, mask=m)
            return idxv + jnp.where(m, L, 0)

        def cmpp4_body(t, idxv):
            j = iA + t * 4
            for u in range(4):
                v = xv[pl.ds(j * L + u * L, L)]
                m = v > cut
                plsc.store_scatter(zv, [idxv], v - mx, mask=m)
                idxv = idxv + jnp.where(m, L, 0)
            return idxv

        def compress():
            idxv = lax.fori_loop(i0, iA, cmpm_body, lane)
            idxv = lax.fori_loop(0, nm4, cmpp4_body, idxv)
            idxv = lax.fori_loop(iA + nm4 * 4, iB, cmpp_body, idxv)
            return lax.fori_loop(iB, i1, cmpm_body, idxv)

        idxv = compress()
        nv = jnp.max((idxv - lane) >> 4)

        @pl.when(nv > KPRE)
        def _refill():
            def clr(r, c):
                zv[pl.ds(r * L, L)] = fill
                return c

            lax.fori_loop(0, nv, clr, jnp.int32(0))
            compress()

        z0 = zv[pl.ds(0, L)]

        def bis_body(_, lohi):
            lo, hi = lohi
            t = 0.5 * (lo + hi)
            facc = jnp.maximum(z0 - t, 0.0)

            def f_body(j, acc):
                z = zv[pl.ds(j * L + L, L)]
                return acc + jnp.maximum(z - t, 0.0)

            facc = lax.fori_loop(0, nv - 1, f_body, facc)
            f = jnp.sum(facc) - 1.0
            ok = f >= 0.0
            return (jnp.where(ok, t, lo), jnp.where(ok, hi, t))

        lo, _hi = lax.fori_loop(0, 16, bis_body,
                                (jnp.float32(-1.0), jnp.float32(0.0)))

        m0 = z0 > lo

        def ref_body(j, kS):
            k, S = kS
            z = zv[pl.ds(j * L + L, L)]
            m = z > lo
            return (k + m.astype(jnp.int32), S + jnp.where(m, z, 0.0))

        kacc, Sacc = lax.fori_loop(0, nv - 1, ref_body,
                                   (m0.astype(jnp.int32),
                                    jnp.where(m0, z0, 0.0)))
        k = jnp.sum(kacc)
        S = jnp.sum(Sacc)
        kf = jnp.maximum(k, 1).astype(jnp.float32)
        tauv = (jnp.full((L,), S, jnp.float32) - 1.0) / jnp.full((L,), kf,
                                                                 jnp.float32)
        thr = mx + tauv

        ost = jnp.maximum(st, p0)
        oen = jnp.minimum(en, p0 + CHUNK)
        o0 = ost >> 4
        o1 = (oen + 15) >> 4
        oA = jnp.minimum(o0 + 1, o1)
        oB = jnp.maximum(o1 - 1, oA)

        def outm_body(j, c2):
            g = j * L + lane
            v = xv[pl.ds(j * L, L)]
            m = (g >= ost) & (g < oen)
            prev = ov[pl.ds(j * L - p0, L)]
            ov[pl.ds(j * L - p0, L)] = jnp.where(
                m, jnp.maximum(v - thr, 0.0), prev)
            return c2

        def outp_body(j, c2):
            v = xv[pl.ds(j * L, L)]
            ov[pl.ds(j * L - p0, L)] = jnp.maximum(v - thr, 0.0)
            return c2

        def outp4_body(t, c2):
            j = oA + t * 4
            for u in range(4):
                v = xv[pl.ds(j * L + u * L, L)]
                ov[pl.ds(j * L + u * L - p0, L)] = jnp.maximum(v - thr, 0.0)
            return c2

        no4 = (oB - oA) >> 2
        lax.fori_loop(o0, oA, outm_body, jnp.int32(0))
        lax.fori_loop(0, no4, outp4_body, jnp.int32(0))
        lax.fori_loop(oA + no4 * 4, oB, outp_body, jnp.int32(0))
        lax.fori_loop(oB, o1, outm_body, jnp.int32(0))
        return carry

    lax.fori_loop(s_first, s_last + 1, seg_body, jnp.int32(0))
    pltpu.sync_copy(ov, out_hbm.at[pl.ds(p0, CHUNK)])


@jax.jit
def _run(x, bounds):
    mesh = plsc.VectorSubcoreMesh(core_axis_name="c", subcore_axis_name="s")
    f = functools.partial(
        pl.kernel,
        mesh=mesh,
        out_type=jax.ShapeDtypeStruct((N,), jnp.float32),
        scratch_types=[
            pltpu.VMEM((N,), jnp.float32),
            pltpu.VMEM((N + L,), jnp.float32),
            pltpu.VMEM((CHUNK,), jnp.float32),
            pltpu.VMEM((2 * L,), jnp.int32),
            pltpu.SemaphoreType.DMA,
        ],
        compiler_params=pltpu.CompilerParams(needs_layout_passes=False,
                                             disable_bounds_checks=True),
    )(_sc_body)
    return f(x, bounds)


def kernel(x, batch):
    batch = batch.astype(jnp.int32)
    ids = jnp.arange(NSEG + 1, dtype=jnp.int32)
    cnt = jnp.sum((ids[:, None] > batch[None, :]).astype(jnp.int32), axis=1)
    bounds = jnp.concatenate([cnt[:NSEG], cnt[1:]]).astype(jnp.int32)
    return _run(x.astype(jnp.float32), bounds)
